# baseline (speedup 1.0000x reference)
import jax
import jax.numpy as jnp
from jax.experimental import pallas as pl


def _copy_block(x_ref, o_ref):
    o_ref[...] = x_ref[...]


def kernel(embedding_weight):
    K, D = embedding_weight.shape
    BK = 1024
    return pl.pallas_call(
        _copy_block,
        grid=(K // BK,),
        in_specs=[pl.BlockSpec((BK, D), lambda i: (i, 0))],
        out_specs=pl.BlockSpec((BK, D), lambda i: (i, 0)),
        out_shape=jax.ShapeDtypeStruct((K, D), embedding_weight.dtype),
    )(embedding_weight)
